# Initial kernel scaffold; baseline (speedup 1.0000x reference)
import jax
import jax.numpy as jnp
from jax.experimental import pallas as pl


def kernel(input0, input1, input2, plat0_w, plat0_b, plat1_w, plat1_b, plat2_w, plat2_b, L0_lat1_0_dw, L0_lat1_0_pw, L0_lat1_0_bn_scale, L0_lat1_0_bn_shift, L0_lat1_1_dw, L0_lat1_1_pw, L0_lat1_1_bn_scale, L0_lat1_1_bn_shift, L0_lat2_0_dw, L0_lat2_0_pw, L0_lat2_0_bn_scale, L0_lat2_0_bn_shift, L0_lat2_1_dw, L0_lat2_1_pw, L0_lat2_1_bn_scale, L0_lat2_1_bn_shift, L0_wtd_0, L0_wtd_1, L0_wbu_0, L0_wbu_1, L1_lat1_0_dw, L1_lat1_0_pw, L1_lat1_0_bn_scale, L1_lat1_0_bn_shift, L1_lat1_1_dw, L1_lat1_1_pw, L1_lat1_1_bn_scale, L1_lat1_1_bn_shift, L1_lat2_0_dw, L1_lat2_0_pw, L1_lat2_0_bn_scale, L1_lat2_0_bn_shift, L1_lat2_1_dw, L1_lat2_1_pw, L1_lat2_1_bn_scale, L1_lat2_1_bn_shift, L1_wtd_0, L1_wtd_1, L1_wbu_0, L1_wbu_1):
    raise NotImplementedError("write your pallas kernel here")


# speedup vs baseline: 11.9601x; 11.9601x over previous
import jax
import jax.numpy as jnp
from jax.experimental import pallas as pl
from jax.experimental.pallas import tpu as pltpu

_EPS_FUSED = 1e-4


def _mm(x, w):
    h, g, k = x.shape
    y = jnp.dot(x.reshape(h * g, k), w, preferred_element_type=jnp.float32)
    return y.reshape(h, g, 128)


def _up2(x):
    h, g, _ = x.shape
    xh = jnp.broadcast_to(x[:, None], (h, 2, g, 128)).reshape(2 * h, g, 128)
    e = jnp.concatenate(
        [xh[..., 0:32], xh[..., 0:32], xh[..., 32:64], xh[..., 32:64]], axis=-1)
    o = jnp.concatenate(
        [xh[..., 64:96], xh[..., 64:96], xh[..., 96:128], xh[..., 96:128]],
        axis=-1)
    return jnp.stack([e, o], axis=2).reshape(2 * h, 2 * g, 128)


def _down2(x):
    h, g, _ = x.shape
    xh = x.reshape(h // 2, 2, g, 128)[:, 0]
    ab = xh.reshape(h // 2, g // 2, 2, 128)
    a = ab[:, :, 0]
    b = ab[:, :, 1]
    return jnp.concatenate(
        [a[..., 0:32], a[..., 64:96], b[..., 0:32], b[..., 64:96]], axis=-1)


def _bifpn_body(wn_ref, x0_ref, x1_ref, x2_ref, lw0_ref, lw1_ref, lw2_ref,
                lb_ref, pws_ref, dws_ref, bns_ref, bnt_ref,
                o0_ref, o1_ref, o2_ref):
    def fused(plist, k):
        acc = wn_ref[k, 0] * plist[0]
        for j in range(1, len(plist)):
            acc = acc + wn_ref[k, j] * plist[j]
        x = acc * dws_ref[k:k + 1, :]
        y = _mm(x, pws_ref[128 * k:128 * (k + 1), :])
        y = y * bns_ref[k:k + 1, :] + bnt_ref[k:k + 1, :]
        return jnp.maximum(y, 0.0)

    p0 = _mm(x0_ref[0], lw0_ref[...]) + lb_ref[0:1, :]
    p1 = _mm(x1_ref[0], lw1_ref[...]) + lb_ref[1:2, :]
    p2 = _mm(x2_ref[0], lw2_ref[...]) + lb_ref[2:3, :]

    ps = [p2, p1, p0]
    for l in range(2):
        base = 4 * l
        a2 = ps[0]
        a1 = fused([ps[1], _up2(a2)], base + 0)
        a0 = fused([ps[2], _up2(a1)], base + 1)
        o1 = fused([ps[1], a1, _down2(a0)], base + 2)
        o2 = fused([ps[0], a2, _down2(o1)], base + 3)
        ps = [o2, o1, a0]

    o2_ref[0] = ps[0]
    o1_ref[0] = ps[1]
    o0_ref[0] = ps[2]


def _kron4(m):
    return jnp.kron(jnp.eye(4, dtype=m.dtype), m)


def _tile4(v):
    return jnp.tile(v, 4)


def _wn(w_raw):
    w = jnp.maximum(w_raw, 0.0)
    w = w / (jnp.sum(w) + _EPS_FUSED)
    return jnp.pad(w, (0, 3 - w.shape[0]))


def kernel(input0, input1, input2, plat0_w, plat0_b, plat1_w, plat1_b,
           plat2_w, plat2_b, L0_lat1_0_dw, L0_lat1_0_pw, L0_lat1_0_bn_scale,
           L0_lat1_0_bn_shift, L0_lat1_1_dw, L0_lat1_1_pw, L0_lat1_1_bn_scale,
           L0_lat1_1_bn_shift, L0_lat2_0_dw, L0_lat2_0_pw, L0_lat2_0_bn_scale,
           L0_lat2_0_bn_shift, L0_lat2_1_dw, L0_lat2_1_pw, L0_lat2_1_bn_scale,
           L0_lat2_1_bn_shift, L0_wtd_0, L0_wtd_1, L0_wbu_0, L0_wbu_1,
           L1_lat1_0_dw, L1_lat1_0_pw, L1_lat1_0_bn_scale, L1_lat1_0_bn_shift,
           L1_lat1_1_dw, L1_lat1_1_pw, L1_lat1_1_bn_scale, L1_lat1_1_bn_shift,
           L1_lat2_0_dw, L1_lat2_0_pw, L1_lat2_0_bn_scale, L1_lat2_0_bn_shift,
           L1_lat2_1_dw, L1_lat2_1_pw, L1_lat2_1_bn_scale, L1_lat2_1_bn_shift,
           L1_wtd_0, L1_wtd_1, L1_wbu_0, L1_wbu_1):
    n = input0.shape[0]

    def prep_x(x):
        _, c, h, w = x.shape
        return jnp.transpose(x, (0, 2, 3, 1)).reshape(n, h, w // 4, 4 * c)

    x0 = prep_x(input0)
    x1 = prep_x(input1)
    x2 = prep_x(input2)

    lw0 = _kron4(plat0_w)
    lw1 = _kron4(plat1_w)
    lw2 = _kron4(plat2_w)
    lb = jnp.stack([_tile4(plat0_b), _tile4(plat1_b), _tile4(plat2_b)])

    steps = [
        (L0_lat1_0_dw, L0_lat1_0_pw, L0_lat1_0_bn_scale, L0_lat1_0_bn_shift, L0_wtd_0),
        (L0_lat1_1_dw, L0_lat1_1_pw, L0_lat1_1_bn_scale, L0_lat1_1_bn_shift, L0_wtd_1),
        (L0_lat2_0_dw, L0_lat2_0_pw, L0_lat2_0_bn_scale, L0_lat2_0_bn_shift, L0_wbu_0),
        (L0_lat2_1_dw, L0_lat2_1_pw, L0_lat2_1_bn_scale, L0_lat2_1_bn_shift, L0_wbu_1),
        (L1_lat1_0_dw, L1_lat1_0_pw, L1_lat1_0_bn_scale, L1_lat1_0_bn_shift, L1_wtd_0),
        (L1_lat1_1_dw, L1_lat1_1_pw, L1_lat1_1_bn_scale, L1_lat1_1_bn_shift, L1_wtd_1),
        (L1_lat2_0_dw, L1_lat2_0_pw, L1_lat2_0_bn_scale, L1_lat2_0_bn_shift, L1_wbu_0),
        (L1_lat2_1_dw, L1_lat2_1_pw, L1_lat2_1_bn_scale, L1_lat2_1_bn_shift, L1_wbu_1),
    ]
    pws = jnp.concatenate([_kron4(s[1]) for s in steps], axis=0)
    dws = jnp.stack([_tile4(s[0]) for s in steps])
    bns = jnp.stack([_tile4(s[2]) for s in steps])
    bnt = jnp.stack([_tile4(s[3]) for s in steps])
    wn = jnp.stack([_wn(s[4]) for s in steps])

    const = lambda i, w_: (0, 0)
    o0, o1, o2 = pl.pallas_call(
        _bifpn_body,
        out_shape=[
            jax.ShapeDtypeStruct((n, 128, 32, 128), jnp.float32),
            jax.ShapeDtypeStruct((n, 64, 16, 128), jnp.float32),
            jax.ShapeDtypeStruct((n, 32, 8, 128), jnp.float32),
        ],
        grid_spec=pltpu.PrefetchScalarGridSpec(
            num_scalar_prefetch=1,
            grid=(n,),
            in_specs=[
                pl.BlockSpec((1, 128, 32, 32), lambda i, w_: (i, 0, 0, 0)),
                pl.BlockSpec((1, 64, 16, 64), lambda i, w_: (i, 0, 0, 0)),
                pl.BlockSpec((1, 32, 8, 96), lambda i, w_: (i, 0, 0, 0)),
                pl.BlockSpec((32, 128), const),
                pl.BlockSpec((64, 128), const),
                pl.BlockSpec((96, 128), const),
                pl.BlockSpec((3, 128), const),
                pl.BlockSpec((1024, 128), const),
                pl.BlockSpec((8, 128), const),
                pl.BlockSpec((8, 128), const),
                pl.BlockSpec((8, 128), const),
            ],
            out_specs=[
                pl.BlockSpec((1, 128, 32, 128), lambda i, w_: (i, 0, 0, 0)),
                pl.BlockSpec((1, 64, 16, 128), lambda i, w_: (i, 0, 0, 0)),
                pl.BlockSpec((1, 32, 8, 128), lambda i, w_: (i, 0, 0, 0)),
            ],
        ),
        compiler_params=pltpu.CompilerParams(
            dimension_semantics=("parallel",),
            vmem_limit_bytes=64 * 1024 * 1024,
        ),
    )(wn, x0, x1, x2, lw0, lw1, lw2, lb, pws, dws, bns, bnt)

    def unprep(o, h, w):
        return jnp.transpose(o.reshape(n, h, w, 32), (0, 3, 1, 2))

    return [unprep(o2, 32, 32), unprep(o1, 64, 64), unprep(o0, 128, 128)]
